# baseline (speedup 1.0000x reference)
import functools

import jax
import jax.numpy as jnp
from jax import lax
from jax.experimental import pallas as pl
from jax.experimental.pallas import tpu as pltpu
from jax.experimental.pallas import tpu_sc as plsc

EMB_DIM = 64
CHUNK = 128


@functools.cache
def _make_gather(n_rows: int, n_emb: int):
  NC, NS = 2, 16
  NW = NC * NS
  assert n_rows % (NW * CHUNK) == 0
  ch_per_w = n_rows // (NW * CHUNK)
  assert ch_per_w % 2 == 0

  mesh = plsc.VectorSubcoreMesh(core_axis_name="c", subcore_axis_name="s")

  @functools.partial(
      pl.kernel,
      out_type=jax.ShapeDtypeStruct((n_rows, EMB_DIM), jnp.bfloat16),
      mesh=mesh,
      scratch_types=[
          pltpu.VMEM((ch_per_w, CHUNK), jnp.int32),
          pltpu.VMEM((2, CHUNK, EMB_DIM), jnp.float32),
          pltpu.VMEM((2, CHUNK, EMB_DIM), jnp.bfloat16),
          pltpu.SemaphoreType.DMA((2,)),
          pltpu.SemaphoreType.DMA((2,)),
      ],
      compiler_params=pltpu.CompilerParams(
          use_tc_tiling_on_sc=False, needs_layout_passes=False
      ),
  )
  def grab(idx_hbm, table_hbm, out_hbm, idx_v, rows_v, bfout_v, gsem, osem):
    wid = lax.axis_index("s") * NC + lax.axis_index("c")
    base_chunk = wid * ch_per_w
    pltpu.sync_copy(idx_hbm.at[pl.ds(base_chunk, ch_per_w)], idx_v)

    def gather(c, p):
      return pltpu.make_async_copy(
          table_hbm.at[idx_v.at[c]], rows_v.at[p], gsem.at[p]
      )

    def store(c, p):
      return pltpu.make_async_copy(
          bfout_v.at[p],
          out_hbm.at[pl.ds((base_chunk + c) * CHUNK, CHUNK)],
          osem.at[p],
      )

    gather(0, 0).start()
    gather(1, 1).start()

    @pl.loop(0, ch_per_w, step=2)
    def _(c0):
      for p in range(2):
        c = c0 + p
        gather(c, p).wait()

        @pl.when(c >= 2)
        def _():
          store(c - 2, p).wait()

        src = rows_v.at[p]
        dst = bfout_v.at[p]

        even = lax.iota(jnp.int32, 16) * 2

        @pl.loop(0, CHUNK, unroll=4)
        def _(r):
          rr = jnp.full((16,), r, jnp.int32)
          for h in range(2):
            a = plsc.load_gather(src, [rr, even + (h * 32)])
            b = plsc.load_gather(src, [rr, even + (h * 32 + 1)])
            ab = plsc.pack(a, b, format=plsc.PackFormat.INTERLEAVED)
            dst[r, pl.ds(h * 32, 32)] = ab

        store(c, p).start()

        @pl.when(c + 2 < ch_per_w)
        def _():
          gather(c + 2, p).start()

    store(ch_per_w - 2, 0).wait()
    store(ch_per_w - 1, 1).wait()

  return grab


def kernel(input, embedding_weight):
  b, f = input.shape
  n_rows = b * f
  idx = input.astype(jnp.int32).reshape(n_rows // CHUNK, CHUNK)
  grab = _make_gather(n_rows, embedding_weight.shape[0])
  out = grab(idx, embedding_weight)
  return out.reshape(b, f, EMB_DIM)
